# baseline (speedup 1.0000x reference)
import functools

import jax
import jax.numpy as jnp
from jax import lax
from jax.experimental import pallas as pl
from jax.experimental.pallas import tpu as pltpu
from jax.experimental.pallas import tpu_sc as plsc

_B, _N, _C = 2, 2000, 81
_NC, _NS, _L = 2, 16, 16
_NW = _NC * _NS
_G0 = 32
_NU = 2 * _G0
_KMAX = _NU // _NW
_NV = _N // _L
_NG_TC = _C - 1 - _G0
_EPS_SUM = 0.0001 * (_B * _N * 4 * (_C - 1))


@functools.partial(
    pl.kernel,
    out_type=jax.ShapeDtypeStruct((_NW, _L), jnp.float32),
    mesh=plsc.VectorSubcoreMesh(core_axis_name="c", subcore_axis_name="s"),
    scratch_types=[
        pltpu.VMEM((2, _B, _N), jnp.float32),
        pltpu.VMEM((2, _B, _N), jnp.float32),
        pltpu.VMEM((_B, _N), jnp.float32),
        pltpu.VMEM((2, _B, _N), jnp.float32),
        pltpu.VMEM((2, _B, _N), jnp.float32),
        pltpu.VMEM((_B, _N), jnp.float32),
        pltpu.VMEM((_L,), jnp.float32),
        pltpu.SemaphoreType.DMA,
        pltpu.SemaphoreType.DMA,
    ],
)
def _sc_loss(out_hbm, tgt_hbm, lbl_hbm, res_hbm,
             o_a, t_a, l_a, o_b, t_b, l_b, r_v, sem_a, sem_b):
    wid = lax.axis_index("s") * _NC + lax.axis_index("c")
    lane = lax.iota(jnp.int32, _L)
    gd = lax.GatherDimensionNumbers(
        offset_dims=(), collapsed_slice_dims=(0,), start_index_map=(0,)
    )

    bufs = [(o_a, t_a, l_a, sem_a), (o_b, t_b, l_b, sem_b)]

    def copies(k, slot):
        u = wid + k * _NW
        o_v, t_v, l_v, sem = bufs[slot]
        return (
            (out_hbm.at[pl.ds(4 + 2 * u, 2), :, :], o_v, sem),
            (tgt_hbm.at[pl.ds(4 + 2 * u, 2), :, :], t_v, sem),
            (lbl_hbm.at[1 + (u >> 1)], l_v, sem),
        )

    def issue(k, slot):
        for src, dst, sem in copies(k, slot):
            pltpu.async_copy(src, dst, sem)

    def drain(k, slot):
        for src, dst, sem in copies(k, slot):
            pltpu.make_async_copy(src, dst, sem).wait()

    zero = jnp.zeros((_L,), jnp.float32)

    def process(slot, carry):
        o_v, t_v, l_v, _ = bufs[slot]

        def vec_body(v, vc):
            accs = list(vc[:4])
            cnt = vc[4]
            n0 = v * _L
            for b in range(_B):
                lb = l_v[b, pl.ds(n0, _L)]
                pos = jnp.where(lb == 1.0, 1.0, 0.0)
                cnt = cnt + pos
                for j in range(2):
                    o = o_v[j, b, pl.ds(n0, _L)]
                    t = t_v[j, b, pl.ds(n0, _L)]
                    x = jnp.abs(o * pos - t)
                    u = jnp.minimum(x, 1.0)
                    i = 2 * b + j
                    accs[i] = accs[i] + u * ((x + x) - u)
            return (*accs, cnt)

        return lax.fori_loop(0, _NV, vec_body, carry, unroll=2)

    issue(0, 0)
    carry = (zero,) * 5
    for k in range(_KMAX):
        slot = k & 1
        if k + 1 < _KMAX:
            issue(k + 1, slot ^ 1)
        drain(k, slot)
        carry = process(slot, carry)

    acc = (carry[0] + carry[1]) + (carry[2] + carry[3])
    cnt = carry[4]

    def lanesum(x):
        for sh in (8, 4, 2, 1):
            x = x + lax.gather(
                x, (lane ^ sh)[:, None], gd, slice_sizes=(1,),
                mode=lax.GatherScatterMode.PROMISE_IN_BOUNDS,
            )
        return x

    loss_s = lanesum(acc)
    cnt_s = lanesum(cnt)
    is0 = jnp.where(lane == 0, 1.0, 0.0)
    is1 = jnp.where(lane == 1, 1.0, 0.0)
    r_v[...] = loss_s * is0 + cnt_s * is1
    pltpu.sync_copy(r_v, res_hbm.at[wid])


def _tc_body(o_ref, t_ref, l_ref, loss_ref, cnt_ref):
    pos = jnp.where(l_ref[...] == 1.0, 1.0, 0.0)
    x = jnp.abs(o_ref[...] * pos - t_ref[...])
    u = jnp.minimum(x, 1.0)
    pq = u * ((x + x) - u)
    ls = jnp.sum(pq)
    cs = jnp.sum(pos)

    @pl.when(pl.program_id(0) == 0)
    def _():
        loss_ref[0, 0] = 0.0
        cnt_ref[0, 0] = 0.0

    loss_ref[0, 0] += ls
    cnt_ref[0, 0] += cs


_tc_loss = pl.pallas_call(
    _tc_body,
    grid=(_NG_TC,),
    in_specs=[
        pl.BlockSpec((4, _B, _N), lambda i: (_G0 + 1 + i, 0, 0)),
        pl.BlockSpec((4, _B, _N), lambda i: (_G0 + 1 + i, 0, 0)),
        pl.BlockSpec((1, _B, _N), lambda i: (_G0 + 1 + i, 0, 0)),
    ],
    out_specs=[
        pl.BlockSpec(memory_space=pltpu.SMEM),
        pl.BlockSpec(memory_space=pltpu.SMEM),
    ],
    out_shape=[
        jax.ShapeDtypeStruct((1, 1), jnp.float32),
        jax.ShapeDtypeStruct((1, 1), jnp.float32),
    ],
)


@jax.jit
def kernel(output, target, labels_target):
    ot = jnp.transpose(output, (2, 0, 1))
    tt = jnp.transpose(target, (2, 0, 1))
    lt = jnp.transpose(labels_target, (2, 0, 1))
    part = _sc_loss(ot, tt, lt)
    tc2, tc_cnt = _tc_loss(ot, tt, lt)
    loss_sum = 0.5 * (jnp.sum(part[:, 0]) + tc2[0, 0])
    b = jnp.float32(_EPS_SUM) + 2.0 * jnp.sum(part[:, 1]) + 4.0 * tc_cnt[0, 0]
    return loss_sum / b


# speedup vs baseline: 1.5606x; 1.5606x over previous
import functools

import jax
import jax.numpy as jnp
from jax import lax
from jax.experimental import pallas as pl
from jax.experimental.pallas import tpu as pltpu
from jax.experimental.pallas import tpu_sc as plsc

_B, _N, _C = 2, 2000, 81
_NC, _NS, _L = 2, 16, 16
_NW = _NC * _NS
_NU = 2 * (_C - 1)
_KMAX = _NU // _NW
_NV = _N // _L
_EPS_SUM = 0.0001 * (_B * _N * 4 * (_C - 1))


@functools.partial(
    pl.kernel,
    out_type=jax.ShapeDtypeStruct((_NW, _L), jnp.float32),
    mesh=plsc.VectorSubcoreMesh(core_axis_name="c", subcore_axis_name="s"),
    scratch_types=[
        pltpu.VMEM((2, _B, _N), jnp.float32),
        pltpu.VMEM((2, _B, _N), jnp.float32),
        pltpu.VMEM((_B, _N), jnp.float32),
        pltpu.VMEM((2, _B, _N), jnp.float32),
        pltpu.VMEM((2, _B, _N), jnp.float32),
        pltpu.VMEM((_B, _N), jnp.float32),
        pltpu.VMEM((_L,), jnp.float32),
        pltpu.SemaphoreType.DMA,
        pltpu.SemaphoreType.DMA,
    ],
)
def _sc_loss(out_hbm, tgt_hbm, lbl_hbm, res_hbm,
             o_a, t_a, l_a, o_b, t_b, l_b, r_v, sem_a, sem_b):
    wid = lax.axis_index("s") * _NC + lax.axis_index("c")
    lane = lax.iota(jnp.int32, _L)
    gd = lax.GatherDimensionNumbers(
        offset_dims=(), collapsed_slice_dims=(0,), start_index_map=(0,)
    )

    bufs = [(o_a, t_a, l_a, sem_a), (o_b, t_b, l_b, sem_b)]

    def copies(k, slot):
        u = wid + k * _NW
        o_v, t_v, l_v, sem = bufs[slot]
        return (
            (out_hbm.at[pl.ds(4 + 2 * u, 2), :, :], o_v, sem),
            (tgt_hbm.at[pl.ds(4 + 2 * u, 2), :, :], t_v, sem),
            (lbl_hbm.at[1 + (u >> 1)], l_v, sem),
        )

    def issue(k, slot):
        for src, dst, sem in copies(k, slot):
            pltpu.async_copy(src, dst, sem)

    def drain(k, slot):
        for src, dst, sem in copies(k, slot):
            pltpu.make_async_copy(src, dst, sem).wait()

    zero = jnp.zeros((_L,), jnp.float32)

    def process(slot, carry):
        o_v, t_v, l_v, _ = bufs[slot]

        def vec_body(v, vc):
            accs = list(vc[:4])
            cnt = vc[4]
            n0 = v * _L
            for b in range(_B):
                lb = l_v[b, pl.ds(n0, _L)]
                pos = jnp.where(lb == 1.0, 1.0, 0.0)
                cnt = cnt + pos
                for j in range(2):
                    o = o_v[j, b, pl.ds(n0, _L)]
                    t = t_v[j, b, pl.ds(n0, _L)]
                    x = jnp.abs(o * pos - t)
                    u = jnp.minimum(x, 1.0)
                    i = 2 * b + j
                    accs[i] = accs[i] + u * ((x + x) - u)
            return (*accs, cnt)

        return lax.fori_loop(0, _NV, vec_body, carry, unroll=2)

    issue(0, 0)
    carry = (zero,) * 5
    for k in range(_KMAX):
        slot = k & 1
        if k + 1 < _KMAX:
            issue(k + 1, slot ^ 1)
        drain(k, slot)
        carry = process(slot, carry)

    acc = (carry[0] + carry[1]) + (carry[2] + carry[3])
    cnt = carry[4]

    def lanesum(x):
        for sh in (8, 4, 2, 1):
            x = x + lax.gather(
                x, (lane ^ sh)[:, None], gd, slice_sizes=(1,),
                mode=lax.GatherScatterMode.PROMISE_IN_BOUNDS,
            )
        return x

    loss_s = lanesum(acc)
    cnt_s = lanesum(cnt)
    is0 = jnp.where(lane == 0, 1.0, 0.0)
    is1 = jnp.where(lane == 1, 1.0, 0.0)
    r_v[...] = loss_s * is0 + cnt_s * is1
    pltpu.sync_copy(r_v, res_hbm.at[wid])


@jax.jit
def kernel(output, target, labels_target):
    part = _sc_loss(
        jnp.transpose(output, (2, 0, 1)),
        jnp.transpose(target, (2, 0, 1)),
        jnp.transpose(labels_target, (2, 0, 1)),
    )
    loss_sum = 0.5 * jnp.sum(part[:, 0])
    cnt2 = jnp.sum(part[:, 1])
    return loss_sum / (jnp.float32(_EPS_SUM) + 2.0 * cnt2)
